# baseline (speedup 1.0000x reference)
import functools

import jax
import jax.numpy as jnp
from jax import lax
from jax.experimental import pallas as pl
from jax.experimental.pallas import tpu as pltpu
from jax.experimental.pallas import tpu_sc as plsc

_NBUF = 2
_LINE = 128


def _transpose_kernel(v, d, w):
    grid = pl.cdiv(v, w)

    def body(t_ref, o_ref):
        x = t_ref[...]
        for s in range(4):
            o_ref[:, s * d:(s + 1) * d] = x[:, s * 128:(s + 1) * 128].T

    return pl.pallas_call(
        body,
        grid=(grid,),
        in_specs=[pl.BlockSpec((d, w), lambda i: (0, i))],
        out_specs=pl.BlockSpec((w // 4, 4 * d), lambda i: (i, 0)),
        out_shape=jax.ShapeDtypeStruct((grid * (w // 4), 4 * d), jnp.float32),
    )


def _gather_kernel(b, f, d, chunk):
    n_rows = b * f
    fc_per_w = f // 2
    bpw = 1024
    sub = bpw // chunk
    n_chunks = fc_per_w * sub
    mesh = plsc.VectorSubcoreMesh(core_axis_name="c", subcore_axis_name="s")

    @functools.partial(
        pl.kernel,
        mesh=mesh,
        out_type=jax.ShapeDtypeStruct((n_rows, d), jnp.float32),
        scratch_types=[
            pltpu.VMEM((fc_per_w, bpw), jnp.int32),
            [pltpu.VMEM((chunk,), jnp.int32) for _ in range(_NBUF)],
            [pltpu.VMEM((chunk,), jnp.int32) for _ in range(_NBUF)],
            [pltpu.VMEM((chunk, _LINE), jnp.float32) for _ in range(_NBUF)],
            [pltpu.VMEM((chunk, d), jnp.float32) for _ in range(_NBUF)],
            [pltpu.SemaphoreType.DMA for _ in range(_NBUF)],
            [pltpu.SemaphoreType.DMA for _ in range(_NBUF)],
        ],
        compiler_params=pltpu.CompilerParams(use_tc_tiling_on_sc=False, needs_layout_passes=False),
    )
    def k(xt_hbm, t4_hbm, out_hbm, idx_all, lines, dests, rows, comp, sem_g, sem_o):
        cid = lax.axis_index("c")
        sid = lax.axis_index("s")
        fc0 = cid * fc_per_w
        b0 = sid * bpw

        pltpu.sync_copy(
            xt_hbm.at[pl.ds(fc0, fc_per_w), pl.ds(b0, bpw)], idx_all
        )

        lane = lax.iota(jnp.int32, 16)
        lane_f = lane * f

        def fill(i, r):
            j = i // sub
            s = i % sub
            base = (b0 + s * chunk) * f + fc0 + j
            for m in range(chunk // 16):
                iv = idx_all[j, pl.ds(s * chunk + m * 16, 16)]
                lines[r][pl.ds(m * 16, 16)] = ((iv >> 9) << 7) | (iv & 127)
                dests[r][pl.ds(m * 16, 16)] = lane_f + (base + m * 16 * f)

        def gather_start(r):
            pltpu.async_copy(t4_hbm.at[lines[r]], rows[r], sem_g[r])

        def gather_wait(r):
            pltpu.make_async_copy(t4_hbm.at[lines[r]], rows[r], sem_g[r]).wait()

        def store_start(r):
            pltpu.async_copy(comp[r], out_hbm.at[dests[r]], sem_o[r])

        def store_wait(r):
            pltpu.make_async_copy(comp[r], out_hbm.at[dests[r]], sem_o[r]).wait()

        def compact(i, r):
            j = i // sub
            s = i % sub

            def body(m16, _):
                iv = idx_all[j, pl.ds(s * chunk + m16 * 16, 16)]
                ovec = ((iv >> 7) & 3) * d
                rids = m16 * 16 + lane
                for c in range(d):
                    val = plsc.load_gather(rows[r], [rids, ovec + c])
                    plsc.store_scatter(
                        comp[r], [rids, jnp.full((16,), c, jnp.int32)], val
                    )
                return 0

            lax.fori_loop(0, chunk // 16, body, 0)

        for r in range(_NBUF):
            fill(jnp.int32(r), r)
            gather_start(r)

        def ring(g, _):
            for r in range(_NBUF):
                i_old = g * _NBUF + r
                gather_wait(r)
                compact(i_old, r)
                store_start(r)
            for r in range(_NBUF):
                i_new = (g + 1) * _NBUF + r
                store_wait(r)
                fill(i_new, r)
                gather_start(r)
            return 0

        lax.fori_loop(0, n_chunks // _NBUF - 1, ring, 0)

        for r in range(_NBUF):
            i_old = n_chunks - _NBUF + r
            gather_wait(r)
            compact(jnp.int32(i_old), r)
            store_start(r)
        for r in range(_NBUF):
            store_wait(r)

    return k


def kernel(x, table):
    b, f = x.shape
    v, d = table.shape
    t4 = _transpose_kernel(v, d, w=512)(table.T)
    out = _gather_kernel(b, f, d, chunk=256)(x.T, t4)
    return out.reshape(b, f * d)
